# baseline (speedup 1.0000x reference)
import jax
import jax.numpy as jnp
from jax import lax
from jax.experimental import pallas as pl
from jax.experimental.pallas import tpu as pltpu
from jax.experimental.pallas import tpu_sc as plsc

EMBED_DIM = 64
BATCH = 16384
NC, NS = 2, 16
NW = NC * NS
B_PER_W = BATCH // NW
NGRP = B_PER_W // 16
CHUNK = 128
NCHUNK = B_PER_W // CHUNK
NBUF = 3

BLK = 3072
NDG = EMBED_DIM // 8
STREAM_MIN = 32768


def _body_small(idx_ref, table_ref, out_ref, idx_v, *rest):
    rows = rest[:NBUF]
    sem_i = rest[NBUF]
    sem_g = rest[NBUF + 1:2 * NBUF + 1]
    sem_s = rest[2 * NBUF + 1:]

    wid = lax.axis_index("s") * NC + lax.axis_index("c")
    base = wid * B_PER_W

    pltpu.async_copy(idx_ref.at[wid], idx_v, sem_i)
    pltpu.make_async_copy(idx_ref.at[wid], idx_v, sem_i).wait()

    def gather_args(j):
        b = j % NBUF
        return (table_ref.at[idx_v.at[j]], rows[b], sem_g[b])

    def store_args(j):
        b = j % NBUF
        return (rows[b], out_ref.at[pl.ds(base + j * CHUNK, CHUNK)], sem_s[b])

    for j in range(min(NBUF, NCHUNK)):
        pltpu.async_copy(*gather_args(j))
    for j in range(NCHUNK):
        pltpu.make_async_copy(*gather_args(j)).wait()
        pltpu.async_copy(*store_args(j))
        nxt = j + NBUF
        if nxt < NCHUNK:
            pltpu.make_async_copy(*store_args(nxt - NBUF)).wait()
            pltpu.async_copy(*gather_args(nxt))
    for j in range(max(0, NCHUNK - NBUF), NCHUNK):
        pltpu.make_async_copy(*store_args(j)).wait()


def _gather_small(idx, table):
    mesh = plsc.VectorSubcoreMesh(
        core_axis_name="c", subcore_axis_name="s",
        num_cores=NC, num_subcores=NS)
    scratch = [pltpu.VMEM((NCHUNK, CHUNK), jnp.int32)]
    scratch += [pltpu.VMEM((CHUNK, EMBED_DIM), jnp.float32)
                for _ in range(NBUF)]
    scratch += [pltpu.SemaphoreType.DMA for _ in range(1 + 2 * NBUF)]
    return pl.kernel(
        _body_small,
        out_type=jax.ShapeDtypeStruct((BATCH, EMBED_DIM), jnp.float32),
        mesh=mesh,
        compiler_params=pltpu.CompilerParams(use_tc_tiling_on_sc=False),
        scratch_types=scratch,
        name=f"sc_gather_v{table.shape[0]}",
    )(idx.reshape(NW, NCHUNK, CHUNK), table)


def _make_body_stream(V):
    TAIL = (V // 128) * 128
    TW = V - TAIL
    CMAX = ((V - BLK) // 128) * 128

    def body(vs_ref, bs_ref, gmm_ref, tabT_ref, out_ref,
                    vs_v, bs_v, gmm_v, ring, tail_v, stage,
                    sem_l, sem_r0, sem_r1, sem_sc):
        wid = lax.axis_index("s") * NC + lax.axis_index("c")
        iota16 = jax.lax.iota(jnp.int32, 16)
        sem_r = [sem_r0, sem_r1]

        pltpu.async_copy(vs_ref.at[wid], vs_v, sem_l)
        pltpu.async_copy(bs_ref.at[wid], bs_v, sem_l)
        pltpu.async_copy(gmm_ref.at[wid], gmm_v, sem_l)
        pltpu.make_async_copy(vs_ref.at[wid], vs_v, sem_l).wait()
        pltpu.make_async_copy(bs_ref.at[wid], bs_v, sem_l).wait()
        pltpu.make_async_copy(gmm_ref.at[wid], gmm_v, sem_l).wait()

        gmin0 = gmm_v[0, :]
        gmin1 = gmm_v[1, :]
        gmax0 = gmm_v[2, :]
        gmax1 = gmm_v[3, :]

        v_lo = jnp.minimum(vs_v[0, :][0], TAIL - 1)
        v_hi = jnp.minimum(vs_v[NGRP - 1, :][15], TAIL - 1)
        s0 = (v_lo // BLK) * BLK
        nblk = (v_hi - s0) // BLK + 1

        def pcount(mask):
            return plsc.all_reduce_population_count(mask)[0]

        ntail = pcount(gmax0 < TAIL) + pcount(gmax1 < TAIL)

        def blk_start(k):
            return pl.multiple_of(s0 + k * BLK, BLK)

        def blk_cstart(k):
            return pl.multiple_of(jnp.minimum(s0 + k * BLK, CMAX), 128)

        for dg in range(NDG):
            def issue_blk(k, slot, dg=dg):
                pltpu.async_copy(
                    tabT_ref.at[pl.ds(dg * 8, 8), pl.ds(blk_cstart(k), BLK)],
                    ring.at[:, pl.ds(slot * BLK, BLK)],
                    sem_r[slot])

            def wait_blk(k, slot, dg=dg):
                pltpu.make_async_copy(
                    tabT_ref.at[pl.ds(dg * 8, 8), pl.ds(blk_cstart(k), BLK)],
                    ring.at[:, pl.ds(slot * BLK, BLK)],
                    sem_r[slot]).wait()

            pltpu.async_copy(
                tabT_ref.at[pl.ds(dg * 8, 8), pl.ds(TAIL, TW)], tail_v, sem_l)

            issue_blk(0, 0)

            def outer(k, carry, dg=dg, issue_blk=issue_blk, wait_blk=wait_blk):
                nxt = k + 1

                @pl.when(jnp.logical_and(nxt < nblk, (nxt % 2) == 0))
                def _():
                    issue_blk(nxt, 0)

                @pl.when(jnp.logical_and(nxt < nblk, (nxt % 2) == 1))
                def _():
                    issue_blk(nxt, 1)

                @pl.when((k % 2) == 0)
                def _():
                    wait_blk(k, 0)

                @pl.when((k % 2) == 1)
                def _():
                    wait_blk(k, 1)

                start = blk_start(k)
                cstart = blk_cstart(k)
                end_eff = jnp.minimum(start + BLK, TAIL)
                base_col = (k % 2) * BLK - cstart

                glo = pcount(gmax0 < start) + pcount(gmax1 < start)
                ghi = pcount(gmin0 < end_eff) + pcount(gmin1 < end_eff)

                def group(g, c, dg=dg):
                    v16 = vs_v[g, :]
                    mask = jnp.logical_and(v16 >= start, v16 < end_eff)
                    colv = jnp.clip(v16 + base_col, 0, 2 * BLK - 1)
                    rows = g * 16 + iota16
                    for d in range(8):
                        vals = plsc.load_gather(
                            ring, [jnp.full((16,), 1, jnp.int32) * d, colv])
                        plsc.store_scatter(
                            stage, [rows, jnp.full((16,), 1, jnp.int32)
                                    * (dg * 8 + d)], vals, mask=mask)
                    return c

                return lax.fori_loop(glo, ghi, group, carry)

            lax.fori_loop(0, nblk, outer, 0)

            pltpu.make_async_copy(
                tabT_ref.at[pl.ds(dg * 8, 8), pl.ds(TAIL, TW)],
                tail_v, sem_l).wait()

            def tail_group(g, c, dg=dg):
                v16 = vs_v[g, :]
                mask = v16 >= TAIL
                colv = jnp.clip(v16 - TAIL, 0, TW - 1)
                rows = g * 16 + iota16
                for d in range(8):
                    vals = plsc.load_gather(
                        tail_v, [jnp.full((16,), 1, jnp.int32) * d, colv])
                    plsc.store_scatter(
                        stage, [rows, jnp.full((16,), 1, jnp.int32)
                                * (dg * 8 + d)], vals, mask=mask)
                return c

            lax.fori_loop(ntail, NGRP, tail_group, 0)

        for g in range(NGRP):
            pltpu.async_copy(stage.at[pl.ds(g * 16, 16)],
                             out_ref.at[bs_v.at[g]], sem_sc)
        for g in range(NGRP):
            pltpu.make_async_copy(stage.at[pl.ds(g * 16, 16)],
                                  out_ref.at[bs_v.at[g]], sem_sc).wait()

    return body


def _gather_stream(idx, table):
    V = table.shape[0]
    TW = V - (V // 128) * 128
    v_s, b_s = lax.sort_key_val(idx, jnp.arange(BATCH, dtype=jnp.int32))
    vg = v_s.reshape(NW, NGRP, 16)
    gmm = jnp.stack([vg[:, :16, 0], vg[:, 16:, 0],
                     vg[:, :16, 15], vg[:, 16:, 15]], axis=1)
    out = pl.kernel(
        _make_body_stream(V),
        out_type=jax.ShapeDtypeStruct((BATCH, 2 * EMBED_DIM), jnp.float32),
        mesh=plsc.VectorSubcoreMesh(
            core_axis_name="c", subcore_axis_name="s",
            num_cores=NC, num_subcores=NS),
        compiler_params=pltpu.CompilerParams(
            use_tc_tiling_on_sc=True, needs_layout_passes=False),
        scratch_types=[
            pltpu.VMEM((NGRP, 16), jnp.int32),
            pltpu.VMEM((NGRP, 16), jnp.int32),
            pltpu.VMEM((4, 16), jnp.int32),
            pltpu.VMEM((8, 2 * BLK), jnp.float32),
            pltpu.VMEM((8, TW), jnp.float32),
            pltpu.VMEM((B_PER_W, 2 * EMBED_DIM), jnp.float32),
            pltpu.SemaphoreType.DMA,
            pltpu.SemaphoreType.DMA,
            pltpu.SemaphoreType.DMA,
            pltpu.SemaphoreType.DMA,
        ],
        name=f"sc_stream_v{V}",
    )(vg, b_s.reshape(NW, NGRP, 16), gmm, table.T)
    return out[:, :EMBED_DIM]


@jax.jit
def _gather_all(*args):
    idxs = args[:8]
    tables = args[8:]
    outs = []
    for i, (ix, t) in enumerate(zip(idxs, tables)):
        if t.shape[0] >= STREAM_MIN:
            outs.append(_gather_stream(ix, t))
        else:
            outs.append(_gather_small(ix, t))
    return tuple(outs)


def kernel(pat_idx, vis_idx, symp_idx, proc_idx, dis_idx, med_idx, anat_idx,
           pharma_idx, pat_table, vis_table, symp_table, proc_table,
           dis_table, med_table, anat_table, pharma_table):
    outs = _gather_all(
        pat_idx, vis_idx, symp_idx, proc_idx, dis_idx, med_idx, anat_idx,
        pharma_idx, pat_table, vis_table, symp_table, proc_table,
        dis_table, med_table, anat_table, pharma_table)
    x_pat, x_vis, x_symp, x_proc, x_dis, x_med, x_anat, x_pharma = outs
    return (x_pat, x_vis, x_proc, x_dis, x_med, x_symp, x_anat, x_pharma)


# speedup vs baseline: 1.0669x; 1.0669x over previous
import jax
import jax.numpy as jnp
from jax import lax
from jax.experimental import pallas as pl
from jax.experimental.pallas import tpu as pltpu
from jax.experimental.pallas import tpu_sc as plsc

EMBED_DIM = 64
BATCH = 16384
NC, NS = 2, 16
NW = NC * NS
B_PER_W = BATCH // NW
NGRP = B_PER_W // 16
CHUNK = 128
NCHUNK = B_PER_W // CHUNK
NBUF = 3

BLK = 3072
NDG = EMBED_DIM // 8
STREAM_MIN = 500000


def _body_small(idx_ref, table_ref, out_ref, idx_v, *rest):
    rows = rest[:NBUF]
    sem_i = rest[NBUF]
    sem_g = rest[NBUF + 1:2 * NBUF + 1]
    sem_s = rest[2 * NBUF + 1:]

    wid = lax.axis_index("s") * NC + lax.axis_index("c")
    base = wid * B_PER_W

    pltpu.async_copy(idx_ref.at[wid], idx_v, sem_i)
    pltpu.make_async_copy(idx_ref.at[wid], idx_v, sem_i).wait()

    def gather_args(j):
        b = j % NBUF
        return (table_ref.at[idx_v.at[j]], rows[b], sem_g[b])

    def store_args(j):
        b = j % NBUF
        return (rows[b], out_ref.at[pl.ds(base + j * CHUNK, CHUNK)], sem_s[b])

    for j in range(min(NBUF, NCHUNK)):
        pltpu.async_copy(*gather_args(j))
    for j in range(NCHUNK):
        pltpu.make_async_copy(*gather_args(j)).wait()
        pltpu.async_copy(*store_args(j))
        nxt = j + NBUF
        if nxt < NCHUNK:
            pltpu.make_async_copy(*store_args(nxt - NBUF)).wait()
            pltpu.async_copy(*gather_args(nxt))
    for j in range(max(0, NCHUNK - NBUF), NCHUNK):
        pltpu.make_async_copy(*store_args(j)).wait()


def _gather_small(idx, table):
    mesh = plsc.VectorSubcoreMesh(
        core_axis_name="c", subcore_axis_name="s",
        num_cores=NC, num_subcores=NS)
    scratch = [pltpu.VMEM((NCHUNK, CHUNK), jnp.int32)]
    scratch += [pltpu.VMEM((CHUNK, EMBED_DIM), jnp.float32)
                for _ in range(NBUF)]
    scratch += [pltpu.SemaphoreType.DMA for _ in range(1 + 2 * NBUF)]
    return pl.kernel(
        _body_small,
        out_type=jax.ShapeDtypeStruct((BATCH, EMBED_DIM), jnp.float32),
        mesh=mesh,
        compiler_params=pltpu.CompilerParams(use_tc_tiling_on_sc=False),
        scratch_types=scratch,
        name=f"sc_gather_v{table.shape[0]}",
    )(idx.reshape(NW, NCHUNK, CHUNK), table)


def _make_body_stream(V):
    TAIL = (V // 128) * 128
    TW = V - TAIL
    CMAX = ((V - BLK) // 128) * 128

    def body(vs_ref, bs_ref, gmm_ref, tabT_ref, out_ref,
                    vs_v, bs_v, gmm_v, ring, tail_v, stage,
                    sem_l, sem_r0, sem_r1, sem_sc):
        wid = lax.axis_index("s") * NC + lax.axis_index("c")
        iota16 = jax.lax.iota(jnp.int32, 16)
        sem_r = [sem_r0, sem_r1]

        pltpu.async_copy(vs_ref.at[wid], vs_v, sem_l)
        pltpu.async_copy(bs_ref.at[wid], bs_v, sem_l)
        pltpu.async_copy(gmm_ref.at[wid], gmm_v, sem_l)
        pltpu.make_async_copy(vs_ref.at[wid], vs_v, sem_l).wait()
        pltpu.make_async_copy(bs_ref.at[wid], bs_v, sem_l).wait()
        pltpu.make_async_copy(gmm_ref.at[wid], gmm_v, sem_l).wait()

        gmin0 = gmm_v[0, :]
        gmin1 = gmm_v[1, :]
        gmax0 = gmm_v[2, :]
        gmax1 = gmm_v[3, :]

        v_lo = jnp.minimum(vs_v[0, :][0], TAIL - 1)
        v_hi = jnp.minimum(vs_v[NGRP - 1, :][15], TAIL - 1)
        s0 = (v_lo // BLK) * BLK
        nblk = (v_hi - s0) // BLK + 1

        def pcount(mask):
            return plsc.all_reduce_population_count(mask)[0]

        ntail = pcount(gmax0 < TAIL) + pcount(gmax1 < TAIL)

        def blk_start(k):
            return pl.multiple_of(s0 + k * BLK, BLK)

        def blk_cstart(k):
            return pl.multiple_of(jnp.minimum(s0 + k * BLK, CMAX), 128)

        for dg in range(NDG):
            def issue_blk(k, slot, dg=dg):
                pltpu.async_copy(
                    tabT_ref.at[pl.ds(dg * 8, 8), pl.ds(blk_cstart(k), BLK)],
                    ring.at[:, pl.ds(slot * BLK, BLK)],
                    sem_r[slot])

            def wait_blk(k, slot, dg=dg):
                pltpu.make_async_copy(
                    tabT_ref.at[pl.ds(dg * 8, 8), pl.ds(blk_cstart(k), BLK)],
                    ring.at[:, pl.ds(slot * BLK, BLK)],
                    sem_r[slot]).wait()

            pltpu.async_copy(
                tabT_ref.at[pl.ds(dg * 8, 8), pl.ds(TAIL, TW)], tail_v, sem_l)

            issue_blk(0, 0)

            def outer(k, carry, dg=dg, issue_blk=issue_blk, wait_blk=wait_blk):
                nxt = k + 1

                @pl.when(jnp.logical_and(nxt < nblk, (nxt % 2) == 0))
                def _():
                    issue_blk(nxt, 0)

                @pl.when(jnp.logical_and(nxt < nblk, (nxt % 2) == 1))
                def _():
                    issue_blk(nxt, 1)

                @pl.when((k % 2) == 0)
                def _():
                    wait_blk(k, 0)

                @pl.when((k % 2) == 1)
                def _():
                    wait_blk(k, 1)

                start = blk_start(k)
                cstart = blk_cstart(k)
                end_eff = jnp.minimum(start + BLK, TAIL)
                base_col = (k % 2) * BLK - cstart

                glo = pcount(gmax0 < start) + pcount(gmax1 < start)
                ghi = pcount(gmin0 < end_eff) + pcount(gmin1 < end_eff)

                def group(g, c, dg=dg):
                    v16 = vs_v[g, :]
                    mask = jnp.logical_and(v16 >= start, v16 < end_eff)
                    colv = jnp.clip(v16 + base_col, 0, 2 * BLK - 1)
                    rows = g * 16 + iota16
                    for d in range(8):
                        vals = plsc.load_gather(
                            ring, [jnp.full((16,), 1, jnp.int32) * d, colv])
                        plsc.store_scatter(
                            stage, [rows, jnp.full((16,), 1, jnp.int32)
                                    * (dg * 8 + d)], vals, mask=mask)
                    return c

                return lax.fori_loop(glo, ghi, group, carry)

            lax.fori_loop(0, nblk, outer, 0)

            pltpu.make_async_copy(
                tabT_ref.at[pl.ds(dg * 8, 8), pl.ds(TAIL, TW)],
                tail_v, sem_l).wait()

            def tail_group(g, c, dg=dg):
                v16 = vs_v[g, :]
                mask = v16 >= TAIL
                colv = jnp.clip(v16 - TAIL, 0, TW - 1)
                rows = g * 16 + iota16
                for d in range(8):
                    vals = plsc.load_gather(
                        tail_v, [jnp.full((16,), 1, jnp.int32) * d, colv])
                    plsc.store_scatter(
                        stage, [rows, jnp.full((16,), 1, jnp.int32)
                                * (dg * 8 + d)], vals, mask=mask)
                return c

            lax.fori_loop(ntail, NGRP, tail_group, 0)

        for g in range(NGRP):
            pltpu.async_copy(stage.at[pl.ds(g * 16, 16)],
                             out_ref.at[bs_v.at[g]], sem_sc)
        for g in range(NGRP):
            pltpu.make_async_copy(stage.at[pl.ds(g * 16, 16)],
                                  out_ref.at[bs_v.at[g]], sem_sc).wait()

    return body


def _gather_stream(idx, table, after=()):
    V = table.shape[0]
    TW = V - (V // 128) * 128
    v_s, b_s = lax.sort_key_val(idx, jnp.arange(BATCH, dtype=jnp.int32))
    if after:
        v_s, b_s, *_ = lax.optimization_barrier((v_s, b_s) + tuple(after))
    vg = v_s.reshape(NW, NGRP, 16)
    gmm = jnp.stack([vg[:, :16, 0], vg[:, 16:, 0],
                     vg[:, :16, 15], vg[:, 16:, 15]], axis=1)
    out = pl.kernel(
        _make_body_stream(V),
        out_type=jax.ShapeDtypeStruct((BATCH, 2 * EMBED_DIM), jnp.float32),
        mesh=plsc.VectorSubcoreMesh(
            core_axis_name="c", subcore_axis_name="s",
            num_cores=NC, num_subcores=NS),
        compiler_params=pltpu.CompilerParams(
            use_tc_tiling_on_sc=True, needs_layout_passes=False),
        scratch_types=[
            pltpu.VMEM((NGRP, 16), jnp.int32),
            pltpu.VMEM((NGRP, 16), jnp.int32),
            pltpu.VMEM((4, 16), jnp.int32),
            pltpu.VMEM((8, 2 * BLK), jnp.float32),
            pltpu.VMEM((8, TW), jnp.float32),
            pltpu.VMEM((B_PER_W, 2 * EMBED_DIM), jnp.float32),
            pltpu.SemaphoreType.DMA,
            pltpu.SemaphoreType.DMA,
            pltpu.SemaphoreType.DMA,
            pltpu.SemaphoreType.DMA,
        ],
        name=f"sc_stream_v{V}",
    )(vg, b_s.reshape(NW, NGRP, 16), gmm, table.T)
    return out[:, :EMBED_DIM]


@jax.jit
def _gather_all(*args):
    idxs = args[:8]
    tables = args[8:]
    outs = {}
    fast = [i for i, t in enumerate(tables)
            if t.shape[0] < STREAM_MIN and t.shape[0] <= 20000]
    for i in fast:
        outs[i] = _gather_small(idxs[i], tables[i])
    for i, (ix, t) in enumerate(zip(idxs, tables)):
        if i in outs:
            continue
        if t.shape[0] >= STREAM_MIN:
            outs[i] = _gather_stream(ix, t, after=tuple(outs[j] for j in fast))
        else:
            outs[i] = _gather_small(ix, t)
    return tuple(outs[i] for i in range(8))


def kernel(pat_idx, vis_idx, symp_idx, proc_idx, dis_idx, med_idx, anat_idx,
           pharma_idx, pat_table, vis_table, symp_table, proc_table,
           dis_table, med_table, anat_table, pharma_table):
    outs = _gather_all(
        pat_idx, vis_idx, symp_idx, proc_idx, dis_idx, med_idx, anat_idx,
        pharma_idx, pat_table, vis_table, symp_table, proc_table,
        dis_table, med_table, anat_table, pharma_table)
    x_pat, x_vis, x_symp, x_proc, x_dis, x_med, x_anat, x_pharma = outs
    return (x_pat, x_vis, x_proc, x_dis, x_med, x_symp, x_anat, x_pharma)


# speedup vs baseline: 1.1600x; 1.0873x over previous
import jax
import jax.numpy as jnp
from jax import lax
from jax.experimental import pallas as pl
from jax.experimental.pallas import tpu as pltpu
from jax.experimental.pallas import tpu_sc as plsc

EMBED_DIM = 64
BATCH = 16384
NC, NS = 2, 16
NW = NC * NS
B_PER_W = BATCH // NW
NGRP = B_PER_W // 16
CHUNK = 128
NCHUNK = B_PER_W // CHUNK
NBUF = 3

BLK = 2048
NSLOT = 3
NDG = EMBED_DIM // 8
STREAM_MIN = 500000


def _body_small(idx_ref, table_ref, out_ref, idx_v, *rest):
    rows = rest[:NBUF]
    sem_i = rest[NBUF]
    sem_g = rest[NBUF + 1:2 * NBUF + 1]
    sem_s = rest[2 * NBUF + 1:]

    wid = lax.axis_index("s") * NC + lax.axis_index("c")
    base = wid * B_PER_W

    pltpu.async_copy(idx_ref.at[wid], idx_v, sem_i)
    pltpu.make_async_copy(idx_ref.at[wid], idx_v, sem_i).wait()

    def gather_args(j):
        b = j % NBUF
        return (table_ref.at[idx_v.at[j]], rows[b], sem_g[b])

    def store_args(j):
        b = j % NBUF
        return (rows[b], out_ref.at[pl.ds(base + j * CHUNK, CHUNK)], sem_s[b])

    for j in range(min(NBUF, NCHUNK)):
        pltpu.async_copy(*gather_args(j))
    for j in range(NCHUNK):
        pltpu.make_async_copy(*gather_args(j)).wait()
        pltpu.async_copy(*store_args(j))
        nxt = j + NBUF
        if nxt < NCHUNK:
            pltpu.make_async_copy(*store_args(nxt - NBUF)).wait()
            pltpu.async_copy(*gather_args(nxt))
    for j in range(max(0, NCHUNK - NBUF), NCHUNK):
        pltpu.make_async_copy(*store_args(j)).wait()


def _gather_small(idx, table):
    mesh = plsc.VectorSubcoreMesh(
        core_axis_name="c", subcore_axis_name="s",
        num_cores=NC, num_subcores=NS)
    scratch = [pltpu.VMEM((NCHUNK, CHUNK), jnp.int32)]
    scratch += [pltpu.VMEM((CHUNK, EMBED_DIM), jnp.float32)
                for _ in range(NBUF)]
    scratch += [pltpu.SemaphoreType.DMA for _ in range(1 + 2 * NBUF)]
    return pl.kernel(
        _body_small,
        out_type=jax.ShapeDtypeStruct((BATCH, EMBED_DIM), jnp.float32),
        mesh=mesh,
        compiler_params=pltpu.CompilerParams(use_tc_tiling_on_sc=False),
        scratch_types=scratch,
        name=f"sc_gather_v{table.shape[0]}",
    )(idx.reshape(NW, NCHUNK, CHUNK), table)


def _make_body_stream(V):
    TAIL = (V // 128) * 128
    TW = V - TAIL
    CMAX = ((V - BLK) // 128) * 128

    def body(vs_ref, bs_ref, gmm_ref, tabT_ref, out_ref,
                    vs_v, bs_v, gmm_v, ring, tail_v, stage,
                    sem_l, sem_r0, sem_r1, sem_r2, sem_sc):
        wid = lax.axis_index("s") * NC + lax.axis_index("c")
        iota16 = jax.lax.iota(jnp.int32, 16)
        sem_r = [sem_r0, sem_r1, sem_r2]

        pltpu.async_copy(vs_ref.at[wid], vs_v, sem_l)
        pltpu.async_copy(bs_ref.at[wid], bs_v, sem_l)
        pltpu.async_copy(gmm_ref.at[wid], gmm_v, sem_l)
        pltpu.make_async_copy(vs_ref.at[wid], vs_v, sem_l).wait()
        pltpu.make_async_copy(bs_ref.at[wid], bs_v, sem_l).wait()
        pltpu.make_async_copy(gmm_ref.at[wid], gmm_v, sem_l).wait()

        gmin0 = gmm_v[0, :]
        gmin1 = gmm_v[1, :]
        gmax0 = gmm_v[2, :]
        gmax1 = gmm_v[3, :]

        v_lo = jnp.minimum(vs_v[0, :][0], TAIL - 1)
        v_hi = jnp.minimum(vs_v[NGRP - 1, :][15], TAIL - 1)
        s0 = (v_lo // BLK) * BLK
        nblk = (v_hi - s0) // BLK + 1

        def pcount(mask):
            return plsc.all_reduce_population_count(mask)[0]

        ntail = pcount(gmax0 < TAIL) + pcount(gmax1 < TAIL)

        def blk_start(k):
            return pl.multiple_of(s0 + k * BLK, BLK)

        def blk_cstart(k):
            return pl.multiple_of(jnp.minimum(s0 + k * BLK, CMAX), 128)

        for dg in range(NDG):
            def issue_blk(k, slot, dg=dg):
                pltpu.async_copy(
                    tabT_ref.at[pl.ds(dg * 8, 8), pl.ds(blk_cstart(k), BLK)],
                    ring.at[:, pl.ds(slot * BLK, BLK)],
                    sem_r[slot])

            def wait_blk(k, slot, dg=dg):
                pltpu.make_async_copy(
                    tabT_ref.at[pl.ds(dg * 8, 8), pl.ds(blk_cstart(k), BLK)],
                    ring.at[:, pl.ds(slot * BLK, BLK)],
                    sem_r[slot]).wait()

            pltpu.async_copy(
                tabT_ref.at[pl.ds(dg * 8, 8), pl.ds(TAIL, TW)], tail_v, sem_l)

            issue_blk(0, 0)

            @pl.when(1 < nblk)
            def _():
                issue_blk(1, 1)

            def outer(k, carry, dg=dg, issue_blk=issue_blk, wait_blk=wait_blk):
                nxt = k + 2

                for s in range(NSLOT):
                    @pl.when(jnp.logical_and(nxt < nblk, (nxt % NSLOT) == s))
                    def _(s=s):
                        issue_blk(nxt, s)

                for s in range(NSLOT):
                    @pl.when((k % NSLOT) == s)
                    def _(s=s):
                        wait_blk(k, s)

                start = blk_start(k)
                cstart = blk_cstart(k)
                end_eff = jnp.minimum(start + BLK, TAIL)
                base_col = (k % NSLOT) * BLK - cstart

                glo = pcount(gmax0 < start) + pcount(gmax1 < start)
                ghi = pcount(gmin0 < end_eff) + pcount(gmin1 < end_eff)

                def group(g, c, dg=dg):
                    v16 = vs_v[g, :]
                    mask = jnp.logical_and(v16 >= start, v16 < end_eff)
                    colv = jnp.clip(v16 + base_col, 0, NSLOT * BLK - 1)
                    rows = g * 16 + iota16
                    for d in range(8):
                        vals = plsc.load_gather(
                            ring, [jnp.full((16,), 1, jnp.int32) * d, colv])
                        plsc.store_scatter(
                            stage, [rows, jnp.full((16,), 1, jnp.int32)
                                    * (dg * 8 + d)], vals, mask=mask)
                    return c

                return lax.fori_loop(glo, ghi, group, carry)

            lax.fori_loop(0, nblk, outer, 0)

            pltpu.make_async_copy(
                tabT_ref.at[pl.ds(dg * 8, 8), pl.ds(TAIL, TW)],
                tail_v, sem_l).wait()

            def tail_group(g, c, dg=dg):
                v16 = vs_v[g, :]
                mask = v16 >= TAIL
                colv = jnp.clip(v16 - TAIL, 0, TW - 1)
                rows = g * 16 + iota16
                for d in range(8):
                    vals = plsc.load_gather(
                        tail_v, [jnp.full((16,), 1, jnp.int32) * d, colv])
                    plsc.store_scatter(
                        stage, [rows, jnp.full((16,), 1, jnp.int32)
                                * (dg * 8 + d)], vals, mask=mask)
                return c

            lax.fori_loop(ntail, NGRP, tail_group, 0)

        for g in range(NGRP):
            pltpu.async_copy(stage.at[pl.ds(g * 16, 16)],
                             out_ref.at[bs_v.at[g]], sem_sc)
        for g in range(NGRP):
            pltpu.make_async_copy(stage.at[pl.ds(g * 16, 16)],
                                  out_ref.at[bs_v.at[g]], sem_sc).wait()

    return body


def _gather_stream(idx, table, after=()):
    V = table.shape[0]
    TW = V - (V // 128) * 128
    v_s, b_s = lax.sort_key_val(idx, jnp.arange(BATCH, dtype=jnp.int32))
    if after:
        v_s, b_s, *_ = lax.optimization_barrier((v_s, b_s) + tuple(after))
    vg = v_s.reshape(NW, NGRP, 16)
    gmm = jnp.stack([vg[:, :16, 0], vg[:, 16:, 0],
                     vg[:, :16, 15], vg[:, 16:, 15]], axis=1)
    out = pl.kernel(
        _make_body_stream(V),
        out_type=jax.ShapeDtypeStruct((BATCH, 2 * EMBED_DIM), jnp.float32),
        mesh=plsc.VectorSubcoreMesh(
            core_axis_name="c", subcore_axis_name="s",
            num_cores=NC, num_subcores=NS),
        compiler_params=pltpu.CompilerParams(
            use_tc_tiling_on_sc=True, needs_layout_passes=False),
        scratch_types=[
            pltpu.VMEM((NGRP, 16), jnp.int32),
            pltpu.VMEM((NGRP, 16), jnp.int32),
            pltpu.VMEM((4, 16), jnp.int32),
            pltpu.VMEM((8, NSLOT * BLK), jnp.float32),
            pltpu.VMEM((8, TW), jnp.float32),
            pltpu.VMEM((B_PER_W, 2 * EMBED_DIM), jnp.float32),
            pltpu.SemaphoreType.DMA,
            pltpu.SemaphoreType.DMA,
            pltpu.SemaphoreType.DMA,
            pltpu.SemaphoreType.DMA,
            pltpu.SemaphoreType.DMA,
        ],
        name=f"sc_stream_v{V}",
    )(vg, b_s.reshape(NW, NGRP, 16), gmm, table.T)
    return out[:, :EMBED_DIM]


@jax.jit
def _gather_all(*args):
    idxs = args[:8]
    tables = args[8:]
    outs = {}
    fast = [i for i, t in enumerate(tables)
            if t.shape[0] < STREAM_MIN and t.shape[0] <= 20000]
    for i in fast:
        outs[i] = _gather_small(idxs[i], tables[i])
    for i, (ix, t) in enumerate(zip(idxs, tables)):
        if i in outs:
            continue
        if t.shape[0] >= STREAM_MIN:
            outs[i] = _gather_stream(ix, t)
        else:
            outs[i] = _gather_small(ix, t)
    return tuple(outs[i] for i in range(8))


def kernel(pat_idx, vis_idx, symp_idx, proc_idx, dis_idx, med_idx, anat_idx,
           pharma_idx, pat_table, vis_table, symp_table, proc_table,
           dis_table, med_table, anat_table, pharma_table):
    outs = _gather_all(
        pat_idx, vis_idx, symp_idx, proc_idx, dis_idx, med_idx, anat_idx,
        pharma_idx, pat_table, vis_table, symp_table, proc_table,
        dis_table, med_table, anat_table, pharma_table)
    x_pat, x_vis, x_symp, x_proc, x_dis, x_med, x_anat, x_pharma = outs
    return (x_pat, x_vis, x_proc, x_dis, x_med, x_symp, x_anat, x_pharma)


# speedup vs baseline: 1.1629x; 1.0024x over previous
import jax
import jax.numpy as jnp
from jax import lax
from jax.experimental import pallas as pl
from jax.experimental.pallas import tpu as pltpu
from jax.experimental.pallas import tpu_sc as plsc

EMBED_DIM = 64
BATCH = 16384
NC, NS = 2, 16
NW = NC * NS
B_PER_W = BATCH // NW
NGRP = B_PER_W // 16
CHUNK = 128
NCHUNK = B_PER_W // CHUNK
NBUF = 3

BLK = 1536
NSLOT = 4
NDG = EMBED_DIM // 8
STREAM_MIN = 500000


def _body_small(idx_ref, table_ref, out_ref, idx_v, *rest):
    rows = rest[:NBUF]
    sem_i = rest[NBUF]
    sem_g = rest[NBUF + 1:2 * NBUF + 1]
    sem_s = rest[2 * NBUF + 1:]

    wid = lax.axis_index("s") * NC + lax.axis_index("c")
    base = wid * B_PER_W

    pltpu.async_copy(idx_ref.at[wid], idx_v, sem_i)
    pltpu.make_async_copy(idx_ref.at[wid], idx_v, sem_i).wait()

    def gather_args(j):
        b = j % NBUF
        return (table_ref.at[idx_v.at[j]], rows[b], sem_g[b])

    def store_args(j):
        b = j % NBUF
        return (rows[b], out_ref.at[pl.ds(base + j * CHUNK, CHUNK)], sem_s[b])

    for j in range(min(NBUF, NCHUNK)):
        pltpu.async_copy(*gather_args(j))
    for j in range(NCHUNK):
        pltpu.make_async_copy(*gather_args(j)).wait()
        pltpu.async_copy(*store_args(j))
        nxt = j + NBUF
        if nxt < NCHUNK:
            pltpu.make_async_copy(*store_args(nxt - NBUF)).wait()
            pltpu.async_copy(*gather_args(nxt))
    for j in range(max(0, NCHUNK - NBUF), NCHUNK):
        pltpu.make_async_copy(*store_args(j)).wait()


def _gather_small(idx, table):
    mesh = plsc.VectorSubcoreMesh(
        core_axis_name="c", subcore_axis_name="s",
        num_cores=NC, num_subcores=NS)
    scratch = [pltpu.VMEM((NCHUNK, CHUNK), jnp.int32)]
    scratch += [pltpu.VMEM((CHUNK, EMBED_DIM), jnp.float32)
                for _ in range(NBUF)]
    scratch += [pltpu.SemaphoreType.DMA for _ in range(1 + 2 * NBUF)]
    return pl.kernel(
        _body_small,
        out_type=jax.ShapeDtypeStruct((BATCH, EMBED_DIM), jnp.float32),
        mesh=mesh,
        compiler_params=pltpu.CompilerParams(use_tc_tiling_on_sc=False),
        scratch_types=scratch,
        name=f"sc_gather_v{table.shape[0]}",
    )(idx.reshape(NW, NCHUNK, CHUNK), table)


def _make_body_stream(V):
    TAIL = (V // 128) * 128
    TW = V - TAIL
    CMAX = ((V - BLK) // 128) * 128

    def body(vs_ref, bs_ref, gmm_ref, tabT_ref, out_ref,
                    vs_v, bs_v, gmm_v, ring, tail_v, stage,
                    sem_l, sem_r0, sem_r1, sem_r2, sem_r3, sem_sc):
        wid = lax.axis_index("s") * NC + lax.axis_index("c")
        iota16 = jax.lax.iota(jnp.int32, 16)
        sem_r = [sem_r0, sem_r1, sem_r2, sem_r3]

        pltpu.async_copy(vs_ref.at[wid], vs_v, sem_l)
        pltpu.async_copy(bs_ref.at[wid], bs_v, sem_l)
        pltpu.async_copy(gmm_ref.at[wid], gmm_v, sem_l)
        pltpu.make_async_copy(vs_ref.at[wid], vs_v, sem_l).wait()
        pltpu.make_async_copy(bs_ref.at[wid], bs_v, sem_l).wait()
        pltpu.make_async_copy(gmm_ref.at[wid], gmm_v, sem_l).wait()

        gmin0 = gmm_v[0, :]
        gmin1 = gmm_v[1, :]
        gmax0 = gmm_v[2, :]
        gmax1 = gmm_v[3, :]

        v_lo = jnp.minimum(vs_v[0, :][0], TAIL - 1)
        v_hi = jnp.minimum(vs_v[NGRP - 1, :][15], TAIL - 1)
        s0 = (v_lo // BLK) * BLK
        nblk = (v_hi - s0) // BLK + 1

        def pcount(mask):
            return plsc.all_reduce_population_count(mask)[0]

        ntail = pcount(gmax0 < TAIL) + pcount(gmax1 < TAIL)

        def blk_start(k):
            return pl.multiple_of(s0 + k * BLK, BLK)

        def blk_cstart(k):
            return pl.multiple_of(jnp.minimum(s0 + k * BLK, CMAX), 128)

        for dg in range(NDG):
            def issue_blk(k, slot, dg=dg):
                pltpu.async_copy(
                    tabT_ref.at[pl.ds(dg * 8, 8), pl.ds(blk_cstart(k), BLK)],
                    ring.at[:, pl.ds(slot * BLK, BLK)],
                    sem_r[slot])

            def wait_blk(k, slot, dg=dg):
                pltpu.make_async_copy(
                    tabT_ref.at[pl.ds(dg * 8, 8), pl.ds(blk_cstart(k), BLK)],
                    ring.at[:, pl.ds(slot * BLK, BLK)],
                    sem_r[slot]).wait()

            pltpu.async_copy(
                tabT_ref.at[pl.ds(dg * 8, 8), pl.ds(TAIL, TW)], tail_v, sem_l)

            issue_blk(0, 0)

            @pl.when(1 < nblk)
            def _():
                issue_blk(1, 1)

            @pl.when(2 < nblk)
            def _():
                issue_blk(2, 2)

            def outer(k, carry, dg=dg, issue_blk=issue_blk, wait_blk=wait_blk):
                nxt = k + 3

                for s in range(NSLOT):
                    @pl.when(jnp.logical_and(nxt < nblk, (nxt % NSLOT) == s))
                    def _(s=s):
                        issue_blk(nxt, s)

                for s in range(NSLOT):
                    @pl.when((k % NSLOT) == s)
                    def _(s=s):
                        wait_blk(k, s)

                start = blk_start(k)
                cstart = blk_cstart(k)
                end_eff = jnp.minimum(start + BLK, TAIL)
                base_col = (k % NSLOT) * BLK - cstart

                glo = pcount(gmax0 < start) + pcount(gmax1 < start)
                ghi = pcount(gmin0 < end_eff) + pcount(gmin1 < end_eff)

                def group(g, c, dg=dg):
                    v16 = vs_v[g, :]
                    mask = jnp.logical_and(v16 >= start, v16 < end_eff)
                    colv = jnp.clip(v16 + base_col, 0, NSLOT * BLK - 1)
                    rows = g * 16 + iota16
                    for d in range(8):
                        vals = plsc.load_gather(
                            ring, [jnp.full((16,), 1, jnp.int32) * d, colv])
                        plsc.store_scatter(
                            stage, [rows, jnp.full((16,), 1, jnp.int32)
                                    * (dg * 8 + d)], vals, mask=mask)
                    return c

                return lax.fori_loop(glo, ghi, group, carry)

            lax.fori_loop(0, nblk, outer, 0)

            pltpu.make_async_copy(
                tabT_ref.at[pl.ds(dg * 8, 8), pl.ds(TAIL, TW)],
                tail_v, sem_l).wait()

            def tail_group(g, c, dg=dg):
                v16 = vs_v[g, :]
                mask = v16 >= TAIL
                colv = jnp.clip(v16 - TAIL, 0, TW - 1)
                rows = g * 16 + iota16
                for d in range(8):
                    vals = plsc.load_gather(
                        tail_v, [jnp.full((16,), 1, jnp.int32) * d, colv])
                    plsc.store_scatter(
                        stage, [rows, jnp.full((16,), 1, jnp.int32)
                                * (dg * 8 + d)], vals, mask=mask)
                return c

            lax.fori_loop(ntail, NGRP, tail_group, 0)

        for g in range(NGRP):
            pltpu.async_copy(stage.at[pl.ds(g * 16, 16)],
                             out_ref.at[bs_v.at[g]], sem_sc)
        for g in range(NGRP):
            pltpu.make_async_copy(stage.at[pl.ds(g * 16, 16)],
                                  out_ref.at[bs_v.at[g]], sem_sc).wait()

    return body


def _gather_stream(idx, table, after=()):
    V = table.shape[0]
    TW = V - (V // 128) * 128
    v_s, b_s = lax.sort_key_val(idx, jnp.arange(BATCH, dtype=jnp.int32))
    if after:
        v_s, b_s, *_ = lax.optimization_barrier((v_s, b_s) + tuple(after))
    vg = v_s.reshape(NW, NGRP, 16)
    gmm = jnp.stack([vg[:, :16, 0], vg[:, 16:, 0],
                     vg[:, :16, 15], vg[:, 16:, 15]], axis=1)
    out = pl.kernel(
        _make_body_stream(V),
        out_type=jax.ShapeDtypeStruct((BATCH, 2 * EMBED_DIM), jnp.float32),
        mesh=plsc.VectorSubcoreMesh(
            core_axis_name="c", subcore_axis_name="s",
            num_cores=NC, num_subcores=NS),
        compiler_params=pltpu.CompilerParams(
            use_tc_tiling_on_sc=True, needs_layout_passes=False),
        scratch_types=[
            pltpu.VMEM((NGRP, 16), jnp.int32),
            pltpu.VMEM((NGRP, 16), jnp.int32),
            pltpu.VMEM((4, 16), jnp.int32),
            pltpu.VMEM((8, NSLOT * BLK), jnp.float32),
            pltpu.VMEM((8, TW), jnp.float32),
            pltpu.VMEM((B_PER_W, 2 * EMBED_DIM), jnp.float32),
            pltpu.SemaphoreType.DMA,
            pltpu.SemaphoreType.DMA,
            pltpu.SemaphoreType.DMA,
            pltpu.SemaphoreType.DMA,
            pltpu.SemaphoreType.DMA,
            pltpu.SemaphoreType.DMA,
        ],
        name=f"sc_stream_v{V}",
    )(vg, b_s.reshape(NW, NGRP, 16), gmm, table.T)
    return out[:, :EMBED_DIM]


@jax.jit
def _gather_all(*args):
    idxs = args[:8]
    tables = args[8:]
    outs = {}
    fast = [i for i, t in enumerate(tables)
            if t.shape[0] < STREAM_MIN and t.shape[0] <= 20000]
    for i in fast:
        outs[i] = _gather_small(idxs[i], tables[i])
    for i, (ix, t) in enumerate(zip(idxs, tables)):
        if i in outs:
            continue
        if t.shape[0] >= STREAM_MIN:
            outs[i] = _gather_stream(ix, t)
        else:
            outs[i] = _gather_small(ix, t)
    return tuple(outs[i] for i in range(8))


def kernel(pat_idx, vis_idx, symp_idx, proc_idx, dis_idx, med_idx, anat_idx,
           pharma_idx, pat_table, vis_table, symp_table, proc_table,
           dis_table, med_table, anat_table, pharma_table):
    outs = _gather_all(
        pat_idx, vis_idx, symp_idx, proc_idx, dis_idx, med_idx, anat_idx,
        pharma_idx, pat_table, vis_table, symp_table, proc_table,
        dis_table, med_table, anat_table, pharma_table)
    x_pat, x_vis, x_symp, x_proc, x_dis, x_med, x_anat, x_pharma = outs
    return (x_pat, x_vis, x_proc, x_dis, x_med, x_symp, x_anat, x_pharma)


# speedup vs baseline: 1.1799x; 1.0147x over previous
import jax
import jax.numpy as jnp
from jax import lax
from jax.experimental import pallas as pl
from jax.experimental.pallas import tpu as pltpu
from jax.experimental.pallas import tpu_sc as plsc

EMBED_DIM = 64
BATCH = 16384
NC, NS = 2, 16
NW = NC * NS
B_PER_W = BATCH // NW
NGRP = B_PER_W // 16
CHUNK = 128
NCHUNK = B_PER_W // CHUNK
NBUF = 3

BLK = 1024
NSLOT = 6
NDG = EMBED_DIM // 8
STREAM_MIN = 500000


def _body_small(idx_ref, table_ref, out_ref, idx_v, *rest):
    rows = rest[:NBUF]
    sem_i = rest[NBUF]
    sem_g = rest[NBUF + 1:2 * NBUF + 1]
    sem_s = rest[2 * NBUF + 1:]

    wid = lax.axis_index("s") * NC + lax.axis_index("c")
    base = wid * B_PER_W

    pltpu.async_copy(idx_ref.at[wid], idx_v, sem_i)
    pltpu.make_async_copy(idx_ref.at[wid], idx_v, sem_i).wait()

    def gather_args(j):
        b = j % NBUF
        return (table_ref.at[idx_v.at[j]], rows[b], sem_g[b])

    def store_args(j):
        b = j % NBUF
        return (rows[b], out_ref.at[pl.ds(base + j * CHUNK, CHUNK)], sem_s[b])

    for j in range(min(NBUF, NCHUNK)):
        pltpu.async_copy(*gather_args(j))
    for j in range(NCHUNK):
        pltpu.make_async_copy(*gather_args(j)).wait()
        pltpu.async_copy(*store_args(j))
        nxt = j + NBUF
        if nxt < NCHUNK:
            pltpu.make_async_copy(*store_args(nxt - NBUF)).wait()
            pltpu.async_copy(*gather_args(nxt))
    for j in range(max(0, NCHUNK - NBUF), NCHUNK):
        pltpu.make_async_copy(*store_args(j)).wait()


def _gather_small(idx, table):
    mesh = plsc.VectorSubcoreMesh(
        core_axis_name="c", subcore_axis_name="s",
        num_cores=NC, num_subcores=NS)
    scratch = [pltpu.VMEM((NCHUNK, CHUNK), jnp.int32)]
    scratch += [pltpu.VMEM((CHUNK, EMBED_DIM), jnp.float32)
                for _ in range(NBUF)]
    scratch += [pltpu.SemaphoreType.DMA for _ in range(1 + 2 * NBUF)]
    return pl.kernel(
        _body_small,
        out_type=jax.ShapeDtypeStruct((BATCH, EMBED_DIM), jnp.float32),
        mesh=mesh,
        compiler_params=pltpu.CompilerParams(use_tc_tiling_on_sc=False),
        scratch_types=scratch,
        name=f"sc_gather_v{table.shape[0]}",
    )(idx.reshape(NW, NCHUNK, CHUNK), table)


def _make_body_stream(V):
    TAIL = (V // 128) * 128
    TW = V - TAIL
    CMAX = ((V - BLK) // 128) * 128

    def body(vs_ref, bs_ref, gmm_ref, tabT_ref, out_ref,
                    vs_v, bs_v, gmm_v, ring, tail_v, stage,
                    sem_l, sem_r0, sem_r1, sem_r2, sem_r3, sem_r4, sem_r5,
                    sem_sc):
        wid = lax.axis_index("s") * NC + lax.axis_index("c")
        iota16 = jax.lax.iota(jnp.int32, 16)
        sem_r = [sem_r0, sem_r1, sem_r2, sem_r3, sem_r4, sem_r5]

        pltpu.async_copy(vs_ref.at[wid], vs_v, sem_l)
        pltpu.async_copy(bs_ref.at[wid], bs_v, sem_l)
        pltpu.async_copy(gmm_ref.at[wid], gmm_v, sem_l)
        pltpu.make_async_copy(vs_ref.at[wid], vs_v, sem_l).wait()
        pltpu.make_async_copy(bs_ref.at[wid], bs_v, sem_l).wait()
        pltpu.make_async_copy(gmm_ref.at[wid], gmm_v, sem_l).wait()

        gmin0 = gmm_v[0, :]
        gmin1 = gmm_v[1, :]
        gmax0 = gmm_v[2, :]
        gmax1 = gmm_v[3, :]

        v_lo = jnp.minimum(vs_v[0, :][0], TAIL - 1)
        v_hi = jnp.minimum(vs_v[NGRP - 1, :][15], TAIL - 1)
        s0 = (v_lo // BLK) * BLK
        nblk = (v_hi - s0) // BLK + 1

        def pcount(mask):
            return plsc.all_reduce_population_count(mask)[0]

        ntail = pcount(gmax0 < TAIL) + pcount(gmax1 < TAIL)

        def blk_start(k):
            return pl.multiple_of(s0 + k * BLK, BLK)

        def blk_cstart(k):
            return pl.multiple_of(jnp.minimum(s0 + k * BLK, CMAX), 128)

        for dg in range(NDG):
            def issue_blk(k, slot, dg=dg):
                pltpu.async_copy(
                    tabT_ref.at[pl.ds(dg * 8, 8), pl.ds(blk_cstart(k), BLK)],
                    ring.at[:, pl.ds(slot * BLK, BLK)],
                    sem_r[slot])

            def wait_blk(k, slot, dg=dg):
                pltpu.make_async_copy(
                    tabT_ref.at[pl.ds(dg * 8, 8), pl.ds(blk_cstart(k), BLK)],
                    ring.at[:, pl.ds(slot * BLK, BLK)],
                    sem_r[slot]).wait()

            pltpu.async_copy(
                tabT_ref.at[pl.ds(dg * 8, 8), pl.ds(TAIL, TW)], tail_v, sem_l)

            issue_blk(0, 0)

            for p in range(1, NSLOT - 1):
                @pl.when(p < nblk)
                def _(p=p):
                    issue_blk(p, p)

            def outer(k, carry, dg=dg, issue_blk=issue_blk, wait_blk=wait_blk):
                nxt = k + NSLOT - 1

                for s in range(NSLOT):
                    @pl.when(jnp.logical_and(nxt < nblk, (nxt % NSLOT) == s))
                    def _(s=s):
                        issue_blk(nxt, s)

                for s in range(NSLOT):
                    @pl.when((k % NSLOT) == s)
                    def _(s=s):
                        wait_blk(k, s)

                start = blk_start(k)
                cstart = blk_cstart(k)
                end_eff = jnp.minimum(start + BLK, TAIL)
                base_col = (k % NSLOT) * BLK - cstart

                glo = pcount(gmax0 < start) + pcount(gmax1 < start)
                ghi = pcount(gmin0 < end_eff) + pcount(gmin1 < end_eff)

                def group(g, c, dg=dg):
                    v16 = vs_v[g, :]
                    mask = jnp.logical_and(v16 >= start, v16 < end_eff)
                    colv = jnp.clip(v16 + base_col, 0, NSLOT * BLK - 1)
                    rows = g * 16 + iota16
                    for d in range(8):
                        vals = plsc.load_gather(
                            ring, [jnp.full((16,), 1, jnp.int32) * d, colv])
                        plsc.store_scatter(
                            stage, [rows, jnp.full((16,), 1, jnp.int32)
                                    * (dg * 8 + d)], vals, mask=mask)
                    return c

                return lax.fori_loop(glo, ghi, group, carry)

            lax.fori_loop(0, nblk, outer, 0)

            pltpu.make_async_copy(
                tabT_ref.at[pl.ds(dg * 8, 8), pl.ds(TAIL, TW)],
                tail_v, sem_l).wait()

            def tail_group(g, c, dg=dg):
                v16 = vs_v[g, :]
                mask = v16 >= TAIL
                colv = jnp.clip(v16 - TAIL, 0, TW - 1)
                rows = g * 16 + iota16
                for d in range(8):
                    vals = plsc.load_gather(
                        tail_v, [jnp.full((16,), 1, jnp.int32) * d, colv])
                    plsc.store_scatter(
                        stage, [rows, jnp.full((16,), 1, jnp.int32)
                                * (dg * 8 + d)], vals, mask=mask)
                return c

            lax.fori_loop(ntail, NGRP, tail_group, 0)

        for g in range(NGRP):
            pltpu.async_copy(stage.at[pl.ds(g * 16, 16)],
                             out_ref.at[bs_v.at[g]], sem_sc)
        for g in range(NGRP):
            pltpu.make_async_copy(stage.at[pl.ds(g * 16, 16)],
                                  out_ref.at[bs_v.at[g]], sem_sc).wait()

    return body


def _gather_stream(idx, table, after=()):
    V = table.shape[0]
    TW = V - (V // 128) * 128
    v_s, b_s = lax.sort_key_val(idx, jnp.arange(BATCH, dtype=jnp.int32))
    if after:
        v_s, b_s, *_ = lax.optimization_barrier((v_s, b_s) + tuple(after))
    vg = v_s.reshape(NW, NGRP, 16)
    gmm = jnp.stack([vg[:, :16, 0], vg[:, 16:, 0],
                     vg[:, :16, 15], vg[:, 16:, 15]], axis=1)
    out = pl.kernel(
        _make_body_stream(V),
        out_type=jax.ShapeDtypeStruct((BATCH, 2 * EMBED_DIM), jnp.float32),
        mesh=plsc.VectorSubcoreMesh(
            core_axis_name="c", subcore_axis_name="s",
            num_cores=NC, num_subcores=NS),
        compiler_params=pltpu.CompilerParams(
            use_tc_tiling_on_sc=True, needs_layout_passes=False),
        scratch_types=[
            pltpu.VMEM((NGRP, 16), jnp.int32),
            pltpu.VMEM((NGRP, 16), jnp.int32),
            pltpu.VMEM((4, 16), jnp.int32),
            pltpu.VMEM((8, NSLOT * BLK), jnp.float32),
            pltpu.VMEM((8, TW), jnp.float32),
            pltpu.VMEM((B_PER_W, 2 * EMBED_DIM), jnp.float32),
            pltpu.SemaphoreType.DMA,
            pltpu.SemaphoreType.DMA,
            pltpu.SemaphoreType.DMA,
            pltpu.SemaphoreType.DMA,
            pltpu.SemaphoreType.DMA,
            pltpu.SemaphoreType.DMA,
            pltpu.SemaphoreType.DMA,
            pltpu.SemaphoreType.DMA,
        ],
        name=f"sc_stream_v{V}",
    )(vg, b_s.reshape(NW, NGRP, 16), gmm, table.T)
    return out[:, :EMBED_DIM]


@jax.jit
def _gather_all(*args):
    idxs = args[:8]
    tables = args[8:]
    outs = {}
    fast = [i for i, t in enumerate(tables)
            if t.shape[0] < STREAM_MIN and t.shape[0] <= 20000]
    for i in fast:
        outs[i] = _gather_small(idxs[i], tables[i])
    for i, (ix, t) in enumerate(zip(idxs, tables)):
        if i in outs:
            continue
        if t.shape[0] >= STREAM_MIN:
            outs[i] = _gather_stream(ix, t)
        else:
            outs[i] = _gather_small(ix, t)
    return tuple(outs[i] for i in range(8))


def kernel(pat_idx, vis_idx, symp_idx, proc_idx, dis_idx, med_idx, anat_idx,
           pharma_idx, pat_table, vis_table, symp_table, proc_table,
           dis_table, med_table, anat_table, pharma_table):
    outs = _gather_all(
        pat_idx, vis_idx, symp_idx, proc_idx, dis_idx, med_idx, anat_idx,
        pharma_idx, pat_table, vis_table, symp_table, proc_table,
        dis_table, med_table, anat_table, pharma_table)
    x_pat, x_vis, x_symp, x_proc, x_dis, x_med, x_anat, x_pharma = outs
    return (x_pat, x_vis, x_proc, x_dis, x_med, x_symp, x_anat, x_pharma)
